# Initial kernel scaffold; baseline (speedup 1.0000x reference)
import jax
import jax.numpy as jnp
from jax.experimental import pallas as pl


def kernel(relative_positions, embeddings_table):
    raise NotImplementedError("write your pallas kernel here")


# speedup vs baseline: 3.5789x; 3.5789x over previous
import functools

import jax
import jax.numpy as jnp
from jax import lax
from jax.experimental import pallas as pl
from jax.experimental.pallas import tpu as pltpu
from jax.experimental.pallas import tpu_sc as plsc

NUM_UNITS = 64
MAX_REL = 128
TABLE_ROWS = 2 * MAX_REL + 1
SEQ = 2048
B = SEQ * SEQ

NC = 2
NS = 16
NW = NC * NS
LANES = 16

CHUNK = 1024
B_PER_W = B // NW
N_ITERS = B_PER_W // CHUNK


def _body(idx_hbm, table_hbm, out_hbm, idx_v, rows_v, sem):
    wid = lax.axis_index("s") * NC + lax.axis_index("c")
    base = wid * B_PER_W

    def step(i, carry):
        off = base + i * CHUNK
        pltpu.sync_copy(idx_hbm.at[pl.ds(off, CHUNK)], idx_v)
        for g in range(CHUNK // LANES):
            sl = pl.ds(g * LANES, LANES)
            v = idx_v[sl] + MAX_REL
            idx_v[sl] = jnp.minimum(jnp.maximum(v, 0), TABLE_ROWS - 1)
        pltpu.async_copy(table_hbm.at[idx_v], rows_v, sem).wait()
        pltpu.sync_copy(rows_v, out_hbm.at[pl.ds(off, CHUNK)])
        return carry

    lax.fori_loop(0, N_ITERS, step, 0)


@jax.jit
def _run(idx_flat, table):
    mesh = plsc.VectorSubcoreMesh(
        core_axis_name="c", subcore_axis_name="s", num_cores=NC,
        num_subcores=NS)
    return pl.kernel(
        _body,
        out_type=jax.ShapeDtypeStruct((B, NUM_UNITS), jnp.float32),
        mesh=mesh,
        scratch_types=[
            pltpu.VMEM((CHUNK,), jnp.int32),
            pltpu.VMEM((CHUNK, NUM_UNITS), jnp.float32),
            pltpu.SemaphoreType.DMA,
        ],
        compiler_params=pltpu.CompilerParams(use_tc_tiling_on_sc=False),
    )(idx_flat, table)


def kernel(relative_positions, embeddings_table):
    idx_flat = relative_positions.astype(jnp.int32).reshape(B)
    out = _run(idx_flat, embeddings_table)
    return out.reshape(SEQ, SEQ, NUM_UNITS)
